# baseline (speedup 1.0000x reference)
import functools

import jax
import jax.numpy as jnp
from jax import lax
from jax.experimental import pallas as pl
from jax.experimental.pallas import tpu as pltpu
from jax.experimental.pallas import tpu_sc as plsc

B = 16384
D = 128
DIM = 2 * D
N_SYN = 512
N_SEM = 1024
CTX_GATE_STRENGTH = 2.0
COMMITMENT_COST = 0.25

BM = 512
NB = B // BM

SC_NC = 2
SC_NS = 16
SC_NW = SC_NC * SC_NS
SC_CHUNK = 128
SC_ROWS_PER_W = B // SC_NW
SC_NCHUNK = SC_ROWS_PER_W // SC_CHUNK


def _vq_block(z, a_ref, cs_ref, b_ref, zsum, idx_ref, k):
    mm = lax.dot_general(
        z, a_ref[...], (((1,), (1,)), ((), ())),
        preferred_element_type=jnp.float32,
        precision=lax.Precision.DEFAULT,
    )
    dot = mm[:, :k]
    logits = mm[:, k:] + b_ref[...]
    lmax = jnp.max(logits, axis=1, keepdims=True)
    e = jnp.exp(logits - lmax)
    sm = e * (1.0 / jnp.sum(e, axis=1, keepdims=True))
    d_pure = (zsum + cs_ref[...]) - 2.0 * dot
    d = d_pure - CTX_GATE_STRENGTH * sm
    mind = jnp.min(d, axis=1, keepdims=True)
    mask = d == mind
    iota = lax.broadcasted_iota(jnp.int32, d.shape, 1)
    idx = jnp.min(jnp.where(mask, iota, k), axis=1)
    idx_ref[0, 0, :] = idx
    return jnp.sum(jnp.where(mask, d_pure, 0.0))


def _tc_body(zfr, zfi, zsr, zsi, zsum_f, zsum_s,
             a_syn, a_sem, cs_syn, cs_sem, b_syn, b_sem,
             idx_syn_ref, idx_sem_ref, loss_ref):
    i = pl.program_id(0)

    @pl.when(i == 0)
    def _():
        loss_ref[0, 0] = 0.0

    zf = jnp.concatenate([zfr[...], zfi[...]], axis=1)
    zs = jnp.concatenate([zsr[...], zsi[...]], axis=1)
    acc = _vq_block(zf, a_syn, cs_syn, b_syn, zsum_f[...], idx_syn_ref, N_SYN)
    acc = acc + _vq_block(zs, a_sem, cs_sem, b_sem, zsum_s[...], idx_sem_ref, N_SEM)
    loss_ref[0, 0] += acc


def _tc_quantize(zfr, zfi, zsr, zsi, zsum_f, zsum_s,
                 a_syn, a_sem, cs_syn, cs_sem, b_syn, b_sem):
    zblk = pl.BlockSpec((BM, D), lambda i: (i, 0))
    nblk = pl.BlockSpec((BM, 1), lambda i: (i, 0))
    full = lambda shape: pl.BlockSpec(shape, lambda i: (0, 0))
    return pl.pallas_call(
        _tc_body,
        grid=(NB,),
        in_specs=[
            zblk, zblk, zblk, zblk, nblk, nblk,
            full((2 * N_SYN, DIM)), full((2 * N_SEM, DIM)),
            full((1, N_SYN)), full((1, N_SEM)),
            full((1, N_SYN)), full((1, N_SEM)),
        ],
        out_specs=[
            pl.BlockSpec((1, 1, BM), lambda i: (i, 0, 0)),
            pl.BlockSpec((1, 1, BM), lambda i: (i, 0, 0)),
            pl.BlockSpec(memory_space=pltpu.SMEM, block_shape=(1, 1),
                         index_map=lambda i: (0, 0)),
        ],
        out_shape=[
            jax.ShapeDtypeStruct((NB, 1, BM), jnp.int32),
            jax.ShapeDtypeStruct((NB, 1, BM), jnp.int32),
            jax.ShapeDtypeStruct((1, 1), jnp.float32),
        ],
        compiler_params=pltpu.CompilerParams(
            dimension_semantics=("arbitrary",)),
    )(zfr, zfi, zsr, zsi, zsum_f, zsum_s,
      a_syn, a_sem, cs_syn, cs_sem, b_syn, b_sem)


@functools.cache
def _make_sc_gather2():
    mesh = plsc.VectorSubcoreMesh(core_axis_name="c", subcore_axis_name="s")

    @functools.partial(
        pl.kernel, mesh=mesh,
        out_type=[
            jax.ShapeDtypeStruct((B, DIM), jnp.float32),
            jax.ShapeDtypeStruct((B, DIM), jnp.float32),
        ],
        scratch_types=[
            pltpu.VMEM((2 * SC_NCHUNK, SC_CHUNK), jnp.int32),
            pltpu.VMEM((SC_CHUNK, DIM), jnp.float32),
            pltpu.VMEM((SC_CHUNK, DIM), jnp.float32),
            pltpu.SemaphoreType.DMA,
            pltpu.SemaphoreType.DMA,
        ],
    )
    def gather2(tsyn, tsem, idx_syn_hbm, idx_sem_hbm, out_syn, out_sem,
                idx_v, rows0, rows1, sem0, sem1):
        wid = lax.axis_index("s") * SC_NC + lax.axis_index("c")
        base = wid * SC_ROWS_PER_W
        pltpu.sync_copy(idx_syn_hbm.at[pl.ds(wid * SC_NCHUNK, SC_NCHUNK)],
                        idx_v.at[pl.ds(0, SC_NCHUNK)])
        pltpu.sync_copy(idx_sem_hbm.at[pl.ds(wid * SC_NCHUNK, SC_NCHUNK)],
                        idx_v.at[pl.ds(SC_NCHUNK, SC_NCHUNK)])
        specs = ([(tsyn, out_syn, c, c) for c in range(SC_NCHUNK)]
                 + [(tsem, out_sem, SC_NCHUNK + c, c) for c in range(SC_NCHUNK)])
        bufs = (rows0, rows1)
        sems = (sem0, sem1)

        def start(j):
            t, _, r, _ = specs[j]
            return pltpu.async_copy(t.at[idx_v.at[r]], bufs[j % 2], sems[j % 2])

        cps = [start(0), start(1)]
        for j in range(len(specs)):
            _, out, _, c = specs[j]
            cps[j % 2].wait()
            pltpu.sync_copy(bufs[j % 2],
                            out.at[pl.ds(base + c * SC_CHUNK, SC_CHUNK)])
            if j + 2 < len(specs):
                cps[j % 2] = start(j + 2)

    return gather2


def kernel(z_fast_real, z_fast_imag, z_slow_real, z_slow_imag,
           cb_syn, cb_sem, W_ctx_syn, b_ctx_syn, W_ctx_sem, b_ctx_sem):
    zf_flat = jnp.concatenate([z_fast_real, z_fast_imag], axis=-1)
    zs_flat = jnp.concatenate([z_slow_real, z_slow_imag], axis=-1)
    zsum_f = jnp.sum(zf_flat ** 2, axis=1, keepdims=True)
    zsum_s = jnp.sum(zs_flat ** 2, axis=1, keepdims=True)
    cs_syn = jnp.sum(cb_syn ** 2, axis=1)[None, :]
    cs_sem = jnp.sum(cb_sem ** 2, axis=1)[None, :]
    a_syn = jnp.concatenate([cb_syn, W_ctx_syn], axis=0)
    a_sem = jnp.concatenate([cb_sem, W_ctx_sem], axis=0)

    idx_syn3, idx_sem3, loss_acc = _tc_quantize(
        z_fast_real, z_fast_imag, z_slow_real, z_slow_imag, zsum_f, zsum_s,
        a_syn, a_sem, cs_syn, cs_sem,
        b_ctx_syn[None, :], b_ctx_sem[None, :])

    idx_syn = idx_syn3.reshape(B)
    idx_sem = idx_sem3.reshape(B)

    cb_syn_i = cb_syn.reshape(N_SYN, 2, D).transpose(0, 2, 1).reshape(N_SYN, DIM)
    cb_sem_i = cb_sem.reshape(N_SEM, 2, D).transpose(0, 2, 1).reshape(N_SEM, DIM)

    rows_syn, rows_sem = _make_sc_gather2()(
        cb_syn_i, cb_sem_i,
        idx_syn.reshape(SC_NW * SC_NCHUNK, SC_CHUNK),
        idx_sem.reshape(SC_NW * SC_NCHUNK, SC_CHUNK))

    zq_syn = rows_syn.view(jnp.complex64)
    zq_sem = rows_sem.view(jnp.complex64)
    loss = (1.0 + COMMITMENT_COST) / (B * DIM) * loss_acc[0, 0]
    return (zq_syn, zq_sem, loss, idx_syn, idx_sem)


# speedup vs baseline: 3.1401x; 3.1401x over previous
import functools

import jax
import jax.numpy as jnp
from jax import lax
from jax.experimental import pallas as pl
from jax.experimental.pallas import tpu as pltpu
from jax.experimental.pallas import tpu_sc as plsc

from jax.extend import core as _jex_core
from jax.interpreters import mlir as _mlir
from jax._src.lib.mlir.dialects import hlo as _hlo

_bitcast_c64_p = _jex_core.Primitive("bitcast_f32_pairs_to_c64")


@_bitcast_c64_p.def_abstract_eval
def _bitcast_c64_abstract(x):
    assert x.shape[-1] == 2 and x.dtype == jnp.float32
    return jax.core.ShapedArray(x.shape[:-1], jnp.complex64)


def _bitcast_c64_lowering(ctx, x):
    aval_out = ctx.avals_out[0]
    out_type = _mlir.aval_to_ir_type(ctx.module_context, aval_out)
    return [_hlo.BitcastConvertOp(out_type, x).result]


_mlir.register_lowering(_bitcast_c64_p, _bitcast_c64_lowering)


def _bitcast_to_c64(x):
    return _bitcast_c64_p.bind(x)


B = 16384
D = 128
DIM = 2 * D
N_SYN = 512
N_SEM = 1024
CTX_GATE_STRENGTH = 2.0
COMMITMENT_COST = 0.25

BM = 512
NB = B // BM

SC_NC = 2
SC_NS = 16
SC_NW = SC_NC * SC_NS
SC_CHUNK = 128
SC_ROWS_PER_W = B // SC_NW
SC_NCHUNK = SC_ROWS_PER_W // SC_CHUNK


def _vq_block(z, a_ref, cs_ref, b_ref, zsum, idx_ref, k):
    mm = lax.dot_general(
        z, a_ref[...], (((1,), (1,)), ((), ())),
        preferred_element_type=jnp.float32,
        precision=lax.Precision.DEFAULT,
    )
    dot = mm[:, :k]
    logits = mm[:, k:] + b_ref[...]
    lmax = jnp.max(logits, axis=1, keepdims=True)
    e = jnp.exp(logits - lmax)
    sm = e * (1.0 / jnp.sum(e, axis=1, keepdims=True))
    d_pure = (zsum + cs_ref[...]) - 2.0 * dot
    d = d_pure - CTX_GATE_STRENGTH * sm
    mind = jnp.min(d, axis=1, keepdims=True)
    mask = d == mind
    iota = lax.broadcasted_iota(jnp.int32, d.shape, 1)
    idx = jnp.min(jnp.where(mask, iota, k), axis=1)
    idx_ref[0, 0, :] = idx
    return jnp.sum(jnp.where(mask, d_pure, 0.0))


def _tc_body(zfr, zfi, zsr, zsi, zsum_f, zsum_s,
             a_syn, a_sem, cs_syn, cs_sem, b_syn, b_sem,
             idx_syn_ref, idx_sem_ref, loss_ref):
    i = pl.program_id(0)

    @pl.when(i == 0)
    def _():
        loss_ref[0, 0] = 0.0

    zf = jnp.concatenate([zfr[...], zfi[...]], axis=1)
    zs = jnp.concatenate([zsr[...], zsi[...]], axis=1)
    acc = _vq_block(zf, a_syn, cs_syn, b_syn, zsum_f[...], idx_syn_ref, N_SYN)
    acc = acc + _vq_block(zs, a_sem, cs_sem, b_sem, zsum_s[...], idx_sem_ref, N_SEM)
    loss_ref[0, 0] += acc


def _tc_quantize(zfr, zfi, zsr, zsi, zsum_f, zsum_s,
                 a_syn, a_sem, cs_syn, cs_sem, b_syn, b_sem):
    zblk = pl.BlockSpec((BM, D), lambda i: (i, 0))
    nblk = pl.BlockSpec((BM, 1), lambda i: (i, 0))
    full = lambda shape: pl.BlockSpec(shape, lambda i: (0, 0))
    return pl.pallas_call(
        _tc_body,
        grid=(NB,),
        in_specs=[
            zblk, zblk, zblk, zblk, nblk, nblk,
            full((2 * N_SYN, DIM)), full((2 * N_SEM, DIM)),
            full((1, N_SYN)), full((1, N_SEM)),
            full((1, N_SYN)), full((1, N_SEM)),
        ],
        out_specs=[
            pl.BlockSpec((1, 1, BM), lambda i: (i, 0, 0)),
            pl.BlockSpec((1, 1, BM), lambda i: (i, 0, 0)),
            pl.BlockSpec(memory_space=pltpu.SMEM, block_shape=(1, 1),
                         index_map=lambda i: (0, 0)),
        ],
        out_shape=[
            jax.ShapeDtypeStruct((NB, 1, BM), jnp.int32),
            jax.ShapeDtypeStruct((NB, 1, BM), jnp.int32),
            jax.ShapeDtypeStruct((1, 1), jnp.float32),
        ],
        compiler_params=pltpu.CompilerParams(
            dimension_semantics=("arbitrary",)),
    )(zfr, zfi, zsr, zsi, zsum_f, zsum_s,
      a_syn, a_sem, cs_syn, cs_sem, b_syn, b_sem)


@functools.cache
def _make_sc_gather2():
    mesh = plsc.VectorSubcoreMesh(core_axis_name="c", subcore_axis_name="s")

    @functools.partial(
        pl.kernel, mesh=mesh,
        out_type=[
            jax.ShapeDtypeStruct((B, DIM), jnp.float32),
            jax.ShapeDtypeStruct((B, DIM), jnp.float32),
        ],
        scratch_types=[
            pltpu.VMEM((2 * SC_NCHUNK, SC_CHUNK), jnp.int32),
            pltpu.VMEM((SC_CHUNK, DIM), jnp.float32),
            pltpu.VMEM((SC_CHUNK, DIM), jnp.float32),
            pltpu.SemaphoreType.DMA,
            pltpu.SemaphoreType.DMA,
        ],
    )
    def gather2(tsyn, tsem, idx_syn_hbm, idx_sem_hbm, out_syn, out_sem,
                idx_v, rows0, rows1, sem0, sem1):
        wid = lax.axis_index("s") * SC_NC + lax.axis_index("c")
        base = wid * SC_ROWS_PER_W
        pltpu.sync_copy(idx_syn_hbm.at[pl.ds(wid * SC_NCHUNK, SC_NCHUNK)],
                        idx_v.at[pl.ds(0, SC_NCHUNK)])
        pltpu.sync_copy(idx_sem_hbm.at[pl.ds(wid * SC_NCHUNK, SC_NCHUNK)],
                        idx_v.at[pl.ds(SC_NCHUNK, SC_NCHUNK)])
        specs = ([(tsyn, out_syn, c, c) for c in range(SC_NCHUNK)]
                 + [(tsem, out_sem, SC_NCHUNK + c, c) for c in range(SC_NCHUNK)])
        bufs = (rows0, rows1)
        sems = (sem0, sem1)

        def start(j):
            t, _, r, _ = specs[j]
            return pltpu.async_copy(t.at[idx_v.at[r]], bufs[j % 2], sems[j % 2])

        cps = [start(0), start(1)]
        for j in range(len(specs)):
            _, out, _, c = specs[j]
            cps[j % 2].wait()
            pltpu.sync_copy(bufs[j % 2],
                            out.at[pl.ds(base + c * SC_CHUNK, SC_CHUNK)])
            if j + 2 < len(specs):
                cps[j % 2] = start(j + 2)

    return gather2


def kernel(z_fast_real, z_fast_imag, z_slow_real, z_slow_imag,
           cb_syn, cb_sem, W_ctx_syn, b_ctx_syn, W_ctx_sem, b_ctx_sem):
    zf_flat = jnp.concatenate([z_fast_real, z_fast_imag], axis=-1)
    zs_flat = jnp.concatenate([z_slow_real, z_slow_imag], axis=-1)
    zsum_f = jnp.sum(zf_flat ** 2, axis=1, keepdims=True)
    zsum_s = jnp.sum(zs_flat ** 2, axis=1, keepdims=True)
    cs_syn = jnp.sum(cb_syn ** 2, axis=1)[None, :]
    cs_sem = jnp.sum(cb_sem ** 2, axis=1)[None, :]
    a_syn = jnp.concatenate([cb_syn, W_ctx_syn], axis=0)
    a_sem = jnp.concatenate([cb_sem, W_ctx_sem], axis=0)

    idx_syn3, idx_sem3, loss_acc = _tc_quantize(
        z_fast_real, z_fast_imag, z_slow_real, z_slow_imag, zsum_f, zsum_s,
        a_syn, a_sem, cs_syn, cs_sem,
        b_ctx_syn[None, :], b_ctx_sem[None, :])

    idx_syn = idx_syn3.reshape(B)
    idx_sem = idx_sem3.reshape(B)

    rows_syn, rows_sem = _make_sc_gather2()(
        cb_syn, cb_sem,
        idx_syn.reshape(SC_NW * SC_NCHUNK, SC_CHUNK),
        idx_sem.reshape(SC_NW * SC_NCHUNK, SC_CHUNK))

    j = jnp.complex64(1j)
    zq_syn = rows_syn[:, :D].astype(jnp.complex64) + rows_syn[:, D:].astype(jnp.complex64) * j
    zq_sem = rows_sem[:, :D].astype(jnp.complex64) + rows_sem[:, D:].astype(jnp.complex64) * j
    loss = (1.0 + COMMITMENT_COST) / (B * DIM) * loss_acc[0, 0]
    return (zq_syn, zq_sem, loss, idx_syn, idx_sem)


# speedup vs baseline: 3.1501x; 1.0032x over previous
import functools

import jax
import jax.numpy as jnp
from jax import lax
from jax.experimental import pallas as pl
from jax.experimental.pallas import tpu as pltpu
from jax.experimental.pallas import tpu_sc as plsc

from jax.extend import core as _jex_core
from jax.interpreters import mlir as _mlir
from jax._src.lib.mlir.dialects import hlo as _hlo

_bitcast_c64_p = _jex_core.Primitive("bitcast_f32_pairs_to_c64")


@_bitcast_c64_p.def_abstract_eval
def _bitcast_c64_abstract(x):
    assert x.shape[-1] == 2 and x.dtype == jnp.float32
    return jax.core.ShapedArray(x.shape[:-1], jnp.complex64)


def _bitcast_c64_lowering(ctx, x):
    aval_out = ctx.avals_out[0]
    out_type = _mlir.aval_to_ir_type(ctx.module_context, aval_out)
    return [_hlo.BitcastConvertOp(out_type, x).result]


_mlir.register_lowering(_bitcast_c64_p, _bitcast_c64_lowering)


def _bitcast_to_c64(x):
    return _bitcast_c64_p.bind(x)


B = 16384
D = 128
DIM = 2 * D
N_SYN = 512
N_SEM = 1024
CTX_GATE_STRENGTH = 2.0
COMMITMENT_COST = 0.25

BM = 512
NB = B // BM

SC_NC = 2
SC_NS = 16
SC_NW = SC_NC * SC_NS
SC_CHUNK = 128
SC_ROWS_PER_W = B // SC_NW
SC_NCHUNK = SC_ROWS_PER_W // SC_CHUNK


def _vq_block(z, a_ref, cs_ref, b_ref, zsum, idx_ref, k):
    mm = lax.dot_general(
        z, a_ref[...], (((1,), (1,)), ((), ())),
        preferred_element_type=jnp.float32,
        precision=lax.Precision.DEFAULT,
    )
    dot = mm[:, :k]
    logits = mm[:, k:] + b_ref[...]
    lmax = jnp.max(logits, axis=1, keepdims=True)
    e = jnp.exp(logits - lmax)
    sm = e * (1.0 / jnp.sum(e, axis=1, keepdims=True))
    d_pure = (zsum + cs_ref[...]) - 2.0 * dot
    d = d_pure - CTX_GATE_STRENGTH * sm
    mind = jnp.min(d, axis=1, keepdims=True)
    mask = d == mind
    iota = lax.broadcasted_iota(jnp.int32, d.shape, 1)
    idx = jnp.min(jnp.where(mask, iota, k), axis=1)
    idx_ref[0, 0, :] = idx
    return jnp.sum(jnp.where(mask, d_pure, 0.0))


def _tc_body(zfr, zfi, zsr, zsi, zsum_f, zsum_s,
             a_syn, a_sem, cs_syn, cs_sem, b_syn, b_sem,
             idx_syn_ref, idx_sem_ref, loss_ref):
    i = pl.program_id(0)

    @pl.when(i == 0)
    def _():
        loss_ref[0, 0] = 0.0

    zf = jnp.concatenate([zfr[...], zfi[...]], axis=1)
    zs = jnp.concatenate([zsr[...], zsi[...]], axis=1)
    acc = _vq_block(zf, a_syn, cs_syn, b_syn, zsum_f[...], idx_syn_ref, N_SYN)
    acc = acc + _vq_block(zs, a_sem, cs_sem, b_sem, zsum_s[...], idx_sem_ref, N_SEM)
    loss_ref[0, 0] += acc


def _tc_quantize(zfr, zfi, zsr, zsi, zsum_f, zsum_s,
                 a_syn, a_sem, cs_syn, cs_sem, b_syn, b_sem):
    zblk = pl.BlockSpec((BM, D), lambda i: (i, 0))
    nblk = pl.BlockSpec((BM, 1), lambda i: (i, 0))
    full = lambda shape: pl.BlockSpec(shape, lambda i: (0, 0))
    return pl.pallas_call(
        _tc_body,
        grid=(NB,),
        in_specs=[
            zblk, zblk, zblk, zblk, nblk, nblk,
            full((2 * N_SYN, DIM)), full((2 * N_SEM, DIM)),
            full((1, N_SYN)), full((1, N_SEM)),
            full((1, N_SYN)), full((1, N_SEM)),
        ],
        out_specs=[
            pl.BlockSpec((1, 1, BM), lambda i: (i, 0, 0)),
            pl.BlockSpec((1, 1, BM), lambda i: (i, 0, 0)),
            pl.BlockSpec(memory_space=pltpu.SMEM, block_shape=(1, 1),
                         index_map=lambda i: (0, 0)),
        ],
        out_shape=[
            jax.ShapeDtypeStruct((NB, 1, BM), jnp.int32),
            jax.ShapeDtypeStruct((NB, 1, BM), jnp.int32),
            jax.ShapeDtypeStruct((1, 1), jnp.float32),
        ],
        compiler_params=pltpu.CompilerParams(
            dimension_semantics=("arbitrary",)),
    )(zfr, zfi, zsr, zsi, zsum_f, zsum_s,
      a_syn, a_sem, cs_syn, cs_sem, b_syn, b_sem)


@functools.cache
def _make_sc_gather2():
    mesh = plsc.VectorSubcoreMesh(core_axis_name="c", subcore_axis_name="s")

    @functools.partial(
        pl.kernel, mesh=mesh,
        out_type=[
            jax.ShapeDtypeStruct((B, DIM), jnp.float32),
            jax.ShapeDtypeStruct((B, DIM), jnp.float32),
        ],
        scratch_types=[
            pltpu.VMEM((2 * SC_NCHUNK, SC_CHUNK), jnp.int32),
            pltpu.VMEM((SC_CHUNK, DIM), jnp.float32),
            pltpu.VMEM((SC_CHUNK, DIM), jnp.float32),
            pltpu.VMEM((SC_CHUNK, DIM), jnp.float32),
            pltpu.SemaphoreType.DMA,
            pltpu.SemaphoreType.DMA,
            pltpu.SemaphoreType.DMA,
            pltpu.SemaphoreType.DMA,
            pltpu.SemaphoreType.DMA,
            pltpu.SemaphoreType.DMA,
        ],
    )
    def gather2(tsyn, tsem, idx_syn_hbm, idx_sem_hbm, out_syn, out_sem,
                idx_v, rows0, rows1, rows2, g0, g1, g2, w0, w1, w2):
        wid = lax.axis_index("s") * SC_NC + lax.axis_index("c")
        base = wid * SC_ROWS_PER_W
        pltpu.sync_copy(idx_syn_hbm.at[pl.ds(wid * SC_NCHUNK, SC_NCHUNK)],
                        idx_v.at[pl.ds(0, SC_NCHUNK)])
        pltpu.sync_copy(idx_sem_hbm.at[pl.ds(wid * SC_NCHUNK, SC_NCHUNK)],
                        idx_v.at[pl.ds(SC_NCHUNK, SC_NCHUNK)])
        specs = ([(tsyn, out_syn, c, c) for c in range(SC_NCHUNK)]
                 + [(tsem, out_sem, SC_NCHUNK + c, c) for c in range(SC_NCHUNK)])
        bufs = (rows0, rows1, rows2)
        gsems = (g0, g1, g2)
        wsems = (w0, w1, w2)
        nj = len(specs)

        def start(j):
            t, _, r, _ = specs[j]
            return pltpu.async_copy(t.at[idx_v.at[r]], bufs[j % 3], gsems[j % 3])

        gcps = [start(0), start(1), start(2)]
        wcps = [None, None, None]
        for j in range(nj):
            b = j % 3
            _, out, _, c = specs[j]
            gcps[b].wait()
            wcps[b] = pltpu.async_copy(
                bufs[b], out.at[pl.ds(base + c * SC_CHUNK, SC_CHUNK)], wsems[b])
            if j + 3 < nj:
                wcps[b].wait()
                gcps[b] = start(j + 3)
        for j in range(max(0, nj - 3), nj):
            wcps[j % 3].wait()

    return gather2


def kernel(z_fast_real, z_fast_imag, z_slow_real, z_slow_imag,
           cb_syn, cb_sem, W_ctx_syn, b_ctx_syn, W_ctx_sem, b_ctx_sem):
    zf_flat = jnp.concatenate([z_fast_real, z_fast_imag], axis=-1)
    zs_flat = jnp.concatenate([z_slow_real, z_slow_imag], axis=-1)
    zsum_f = jnp.sum(zf_flat ** 2, axis=1, keepdims=True)
    zsum_s = jnp.sum(zs_flat ** 2, axis=1, keepdims=True)
    cs_syn = jnp.sum(cb_syn ** 2, axis=1)[None, :]
    cs_sem = jnp.sum(cb_sem ** 2, axis=1)[None, :]
    a_syn = jnp.concatenate([cb_syn, W_ctx_syn], axis=0)
    a_sem = jnp.concatenate([cb_sem, W_ctx_sem], axis=0)

    idx_syn3, idx_sem3, loss_acc = _tc_quantize(
        z_fast_real, z_fast_imag, z_slow_real, z_slow_imag, zsum_f, zsum_s,
        a_syn, a_sem, cs_syn, cs_sem,
        b_ctx_syn[None, :], b_ctx_sem[None, :])

    idx_syn = idx_syn3.reshape(B)
    idx_sem = idx_sem3.reshape(B)

    rows_syn, rows_sem = _make_sc_gather2()(
        cb_syn, cb_sem,
        idx_syn.reshape(SC_NW * SC_NCHUNK, SC_CHUNK),
        idx_sem.reshape(SC_NW * SC_NCHUNK, SC_CHUNK))

    j = jnp.complex64(1j)
    zq_syn = rows_syn[:, :D].astype(jnp.complex64) + rows_syn[:, D:].astype(jnp.complex64) * j
    zq_sem = rows_sem[:, :D].astype(jnp.complex64) + rows_sem[:, D:].astype(jnp.complex64) * j
    loss = (1.0 + COMMITMENT_COST) / (B * DIM) * loss_acc[0, 0]
    return (zq_syn, zq_sem, loss, idx_syn, idx_sem)


# speedup vs baseline: 3.2000x; 1.0158x over previous
import functools

import jax
import jax.numpy as jnp
from jax import lax
from jax.experimental import pallas as pl
from jax.experimental.pallas import tpu as pltpu
from jax.experimental.pallas import tpu_sc as plsc

from jax.extend import core as _jex_core
from jax.interpreters import mlir as _mlir
from jax._src.lib.mlir.dialects import hlo as _hlo

_bitcast_c64_p = _jex_core.Primitive("bitcast_f32_pairs_to_c64")


@_bitcast_c64_p.def_abstract_eval
def _bitcast_c64_abstract(x):
    assert x.shape[-1] == 2 and x.dtype == jnp.float32
    return jax.core.ShapedArray(x.shape[:-1], jnp.complex64)


def _bitcast_c64_lowering(ctx, x):
    aval_out = ctx.avals_out[0]
    out_type = _mlir.aval_to_ir_type(ctx.module_context, aval_out)
    return [_hlo.BitcastConvertOp(out_type, x).result]


_mlir.register_lowering(_bitcast_c64_p, _bitcast_c64_lowering)


def _bitcast_to_c64(x):
    return _bitcast_c64_p.bind(x)


B = 16384
D = 128
DIM = 2 * D
N_SYN = 512
N_SEM = 1024
CTX_GATE_STRENGTH = 2.0
COMMITMENT_COST = 0.25

BM = 512
NB = B // BM

SC_NC = 2
SC_NS = 16
SC_NW = SC_NC * SC_NS
SC_CHUNK = 128
SC_ROWS_PER_W = B // SC_NW
SC_NCHUNK = SC_ROWS_PER_W // SC_CHUNK


def _vq_block(z, a_ref, cs_ref, b_ref, zsum, idx_ref, k):
    mm = lax.dot_general(
        z, a_ref[...], (((1,), (1,)), ((), ())),
        preferred_element_type=jnp.float32,
        precision=lax.Precision.DEFAULT,
    )
    dot = mm[:, :k]
    logits = mm[:, k:] + b_ref[...]
    lmax = jnp.max(logits, axis=1, keepdims=True)
    e = jnp.exp(logits - lmax)
    sm = e * (1.0 / jnp.sum(e, axis=1, keepdims=True))
    d_pure = (zsum + cs_ref[...]) - 2.0 * dot
    d = d_pure - CTX_GATE_STRENGTH * sm
    mind = jnp.min(d, axis=1, keepdims=True)
    mask = d == mind
    iota = lax.broadcasted_iota(jnp.int32, d.shape, 1)
    idx = jnp.min(jnp.where(mask, iota, k), axis=1)
    idx_ref[0, 0, :] = idx
    return jnp.sum(jnp.where(mask, d_pure, 0.0))


def _tc_body(k, zr, zi, zsum, a, cs, b, idx_ref, loss_ref):
    i = pl.program_id(0)

    @pl.when(i == 0)
    def _():
        loss_ref[0, 0] = 0.0

    z = jnp.concatenate([zr[...], zi[...]], axis=1)
    loss_ref[0, 0] += _vq_block(z, a, cs, b, zsum[...], idx_ref, k)


def _tc_quantize(k, zr, zi, zsum, a, cs, b):
    zblk = pl.BlockSpec((BM, D), lambda i: (i, 0))
    nblk = pl.BlockSpec((BM, 1), lambda i: (i, 0))
    full = lambda shape: pl.BlockSpec(shape, lambda i: (0, 0))
    return pl.pallas_call(
        functools.partial(_tc_body, k),
        grid=(NB,),
        in_specs=[
            zblk, zblk, nblk,
            full((2 * k, DIM)), full((1, k)), full((1, k)),
        ],
        out_specs=[
            pl.BlockSpec((1, 1, BM), lambda i: (i, 0, 0)),
            pl.BlockSpec(memory_space=pltpu.SMEM, block_shape=(1, 1),
                         index_map=lambda i: (0, 0)),
        ],
        out_shape=[
            jax.ShapeDtypeStruct((NB, 1, BM), jnp.int32),
            jax.ShapeDtypeStruct((1, 1), jnp.float32),
        ],
        compiler_params=pltpu.CompilerParams(
            dimension_semantics=("arbitrary",)),
    )(zr, zi, zsum, a, cs, b)


@functools.cache
def _make_sc_gather(n_codes):
    mesh = plsc.VectorSubcoreMesh(core_axis_name="c", subcore_axis_name="s")

    @functools.partial(
        pl.kernel, mesh=mesh,
        out_type=jax.ShapeDtypeStruct((B, DIM), jnp.float32),
        scratch_types=[
            pltpu.VMEM((SC_NCHUNK, SC_CHUNK), jnp.int32),
            pltpu.VMEM((SC_CHUNK, DIM), jnp.float32),
            pltpu.VMEM((SC_CHUNK, DIM), jnp.float32),
            pltpu.VMEM((SC_CHUNK, DIM), jnp.float32),
            pltpu.SemaphoreType.DMA,
            pltpu.SemaphoreType.DMA,
            pltpu.SemaphoreType.DMA,
            pltpu.SemaphoreType.DMA,
            pltpu.SemaphoreType.DMA,
            pltpu.SemaphoreType.DMA,
        ],
    )
    def gather(table, idx_hbm, out,
               idx_v, rows0, rows1, rows2, g0, g1, g2, w0, w1, w2):
        wid = lax.axis_index("s") * SC_NC + lax.axis_index("c")
        base = wid * SC_ROWS_PER_W
        pltpu.sync_copy(idx_hbm.at[pl.ds(wid * SC_NCHUNK, SC_NCHUNK)], idx_v)
        bufs = (rows0, rows1, rows2)
        gsems = (g0, g1, g2)
        wsems = (w0, w1, w2)
        nj = SC_NCHUNK

        def start(j):
            return pltpu.async_copy(table.at[idx_v.at[j]], bufs[j % 3], gsems[j % 3])

        gcps = [start(0), start(1), start(2)]
        wcps = [None, None, None]
        for j in range(nj):
            b = j % 3
            gcps[b].wait()
            wcps[b] = pltpu.async_copy(
                bufs[b], out.at[pl.ds(base + j * SC_CHUNK, SC_CHUNK)], wsems[b])
            if j + 3 < nj:
                wcps[b].wait()
                gcps[b] = start(j + 3)
        for j in range(max(0, nj - 3), nj):
            wcps[j % 3].wait()

    return gather


def kernel(z_fast_real, z_fast_imag, z_slow_real, z_slow_imag,
           cb_syn, cb_sem, W_ctx_syn, b_ctx_syn, W_ctx_sem, b_ctx_sem):
    zf_flat = jnp.concatenate([z_fast_real, z_fast_imag], axis=-1)
    zs_flat = jnp.concatenate([z_slow_real, z_slow_imag], axis=-1)
    zsum_f = jnp.sum(zf_flat ** 2, axis=1, keepdims=True)
    zsum_s = jnp.sum(zs_flat ** 2, axis=1, keepdims=True)
    cs_syn = jnp.sum(cb_syn ** 2, axis=1)[None, :]
    cs_sem = jnp.sum(cb_sem ** 2, axis=1)[None, :]
    a_syn = jnp.concatenate([cb_syn, W_ctx_syn], axis=0)
    a_sem = jnp.concatenate([cb_sem, W_ctx_sem], axis=0)

    j = jnp.complex64(1j)

    idx_syn3, l_syn = _tc_quantize(
        N_SYN, z_fast_real, z_fast_imag, zsum_f, a_syn, cs_syn,
        b_ctx_syn[None, :])
    idx_syn = idx_syn3.reshape(B)
    rows_syn = _make_sc_gather(N_SYN)(
        cb_syn, idx_syn.reshape(SC_NW * SC_NCHUNK, SC_CHUNK))

    idx_sem3, l_sem = _tc_quantize(
        N_SEM, z_slow_real, z_slow_imag, zsum_s, a_sem, cs_sem,
        b_ctx_sem[None, :])
    idx_sem = idx_sem3.reshape(B)

    zq_syn = (rows_syn[:, :D].astype(jnp.complex64)
              + rows_syn[:, D:].astype(jnp.complex64) * j)

    rows_sem = _make_sc_gather(N_SEM)(
        cb_sem, idx_sem.reshape(SC_NW * SC_NCHUNK, SC_CHUNK))
    zq_sem = (rows_sem[:, :D].astype(jnp.complex64)
              + rows_sem[:, D:].astype(jnp.complex64) * j)

    loss = (1.0 + COMMITMENT_COST) / (B * DIM) * (l_syn[0, 0] + l_sem[0, 0])
    return (zq_syn, zq_sem, loss, idx_syn, idx_sem)
